# Initial kernel scaffold; baseline (speedup 1.0000x reference)
import jax
import jax.numpy as jnp
from jax.experimental import pallas as pl


def kernel(feat, n_node, W1, b1, W2, b2, W3, b3):
    raise NotImplementedError("write your pallas kernel here")


# speedup vs baseline: 42.2527x; 42.2527x over previous
import functools

import jax
import jax.numpy as jnp
from jax.experimental import pallas as pl
from jax.experimental.pallas import tpu as pltpu

LATENT = 128
HIDDEN = 256
OUT_DIM = 1
B_GRAPHS = 500
NPG = 200
G_BLK = 50
N_STEPS = B_GRAPHS // G_BLK


def _fused_kernel(feat_ref, inv_n_ref, w1_ref, b1_ref, w2_ref, b2_ref,
                  w3_ref, b3_ref, out_ref, pooled_ref):
    i = pl.program_id(0)
    x = feat_ref[...]
    s = jnp.sum(x, axis=1)
    pooled_ref[pl.ds(i * G_BLK, G_BLK), :] = s

    @pl.when(i == N_STEPS - 1)
    def _mlp():
        pooled = pooled_ref[...] * inv_n_ref[...]
        h = jnp.maximum(
            jnp.dot(pooled, w1_ref[...], preferred_element_type=jnp.float32)
            + b1_ref[...], 0.0)
        h = jnp.maximum(
            jnp.dot(h, w2_ref[...], preferred_element_type=jnp.float32)
            + b2_ref[...], 0.0)
        out_ref[...] = (
            jnp.dot(h, w3_ref[...], preferred_element_type=jnp.float32)
            + b3_ref[...])


@jax.jit
def kernel(feat, n_node, W1, b1, W2, b2, W3, b3):
    feat3 = feat.reshape(B_GRAPHS, NPG, LATENT)
    inv_n = (1.0 / n_node.astype(jnp.float32))[:, None]
    grid = (N_STEPS,)
    out = pl.pallas_call(
        _fused_kernel,
        grid=grid,
        in_specs=[
            pl.BlockSpec((G_BLK, NPG, LATENT), lambda i: (i, 0, 0)),
            pl.BlockSpec((B_GRAPHS, 1), lambda i: (0, 0)),
            pl.BlockSpec((LATENT, HIDDEN), lambda i: (0, 0)),
            pl.BlockSpec((HIDDEN,), lambda i: (0,)),
            pl.BlockSpec((HIDDEN, HIDDEN), lambda i: (0, 0)),
            pl.BlockSpec((HIDDEN,), lambda i: (0,)),
            pl.BlockSpec((HIDDEN, OUT_DIM), lambda i: (0, 0)),
            pl.BlockSpec((OUT_DIM,), lambda i: (0,)),
        ],
        out_specs=pl.BlockSpec((B_GRAPHS, OUT_DIM), lambda i: (0, 0)),
        out_shape=jax.ShapeDtypeStruct((B_GRAPHS, OUT_DIM), jnp.float32),
        scratch_shapes=[pltpu.VMEM((B_GRAPHS, LATENT), jnp.float32)],
    )(feat3, inv_n, W1, b1, W2, b2, W3, b3)
    return out
